# baseline (speedup 1.0000x reference)
import functools

import jax
import jax.numpy as jnp
from jax import lax
from jax.experimental import pallas as pl
from jax.experimental.pallas import tpu as pltpu
from jax.experimental.pallas import tpu_sc as plsc

H = 64
SENT = 200
BATCH = 4096
TOTAL = BATCH * SENT
EPS = 1e-5
NC = 2
NS = 16
NW = NC * NS
GRP = 128
GPC = 8
CHUNK = GRP * GPC
ROWS_PER_W = TOTAL // NW
NCHUNK = ROWS_PER_W // CHUNK

_mesh = plsc.VectorSubcoreMesh(core_axis_name="c", subcore_axis_name="s")


@functools.partial(
    pl.kernel,
    out_type=jax.ShapeDtypeStruct((TOTAL, H), jnp.float32),
    mesh=_mesh,
    scratch_types=[
        pltpu.VMEM((GPC, GRP), jnp.int32),
        pltpu.VMEM((GPC, GRP, H), jnp.float32),
        pltpu.VMEM((SENT, H), jnp.float32),
        pltpu.VMEM((H,), jnp.float32),
        pltpu.VMEM((H,), jnp.float32),
        pltpu.SemaphoreType.DMA,
    ],
    compiler_params=pltpu.CompilerParams(
        needs_layout_passes=False, use_tc_tiling_on_sc=False),
)
def _sc_embed_ln(ids_hbm, tok_hbm, pos_hbm, w_hbm, b_hbm, out_hbm,
                 idx_v, rows_v, pos_v, w_v, b_v, sem):
    cid = lax.axis_index("c")
    sid = lax.axis_index("s")
    wid = sid * NC + cid
    base_w = wid * ROWS_PER_W

    pltpu.sync_copy(pos_hbm.at[pl.ds(0, SENT)], pos_v)
    pltpu.sync_copy(w_hbm, w_v)
    pltpu.sync_copy(b_hbm, b_v)

    @pl.loop(0, NCHUNK)
    def chunk_loop(g):
        base = base_w + g * CHUNK
        idx_row = pl.multiple_of(base_w // GRP + g * GPC, 8)
        pltpu.sync_copy(ids_hbm.at[pl.ds(idx_row, GPC)], idx_v)
        copies = [
            pltpu.async_copy(tok_hbm.at[idx_v.at[j]], rows_v.at[j], sem)
            for j in range(GPC)
        ]
        for c in copies:
            c.wait()

        for j in range(GPC):
            @plsc.parallel_loop(0, GRP, 1, unroll=2)
            def row_loop(r, _j=j):
                x = []
                p = lax.rem(base + _j * GRP + r, SENT)
                for h in range(4):
                    x.append(rows_v[_j, r, pl.ds(16 * h, 16)]
                             + pos_v[p, pl.ds(16 * h, 16)])
                s = (x[0] + x[1]) + (x[2] + x[3])
                q = (x[0] * x[0] + x[1] * x[1]) + (x[2] * x[2] + x[3] * x[3])
                lanes = lax.iota(jnp.int32, 16)
                for m in (1, 2, 4, 8):
                    perm = lanes ^ m
                    s = s + s.at[perm].get(mode="promise_in_bounds")
                    q = q + q.at[perm].get(mode="promise_in_bounds")
                mv = s * (1.0 / H)
                vv = q * (1.0 / H) - mv * mv + EPS
                iv = plsc.bitcast(vv, jnp.int32)
                y = plsc.bitcast(jnp.int32(0x5F3759DF) - (iv >> 1), jnp.float32)
                hv = vv * 0.5
                y = y * (1.5 - hv * y * y)
                y = y * (1.5 - hv * y * y)
                y = y * (1.5 - hv * y * y)
                for h in range(4):
                    wgt = w_v[pl.ds(16 * h, 16)]
                    bia = b_v[pl.ds(16 * h, 16)]
                    rows_v[_j, r, pl.ds(16 * h, 16)] = (
                        (x[h] - mv) * y * wgt + bia)

        for j in range(GPC):
            pltpu.sync_copy(rows_v.at[j],
                            out_hbm.at[pl.ds(base + j * GRP, GRP)])


def kernel(batch_token_ids, token_table, pos_table, ln_weight, ln_bias):
    b, s = batch_token_ids.shape
    ids2d = batch_token_ids.reshape(TOTAL // GRP, GRP).astype(jnp.int32)
    out = _sc_embed_ln(ids2d, token_table, pos_table, ln_weight, ln_bias)
    return out.reshape(b, s, H)


# speedup vs baseline: 1.0503x; 1.0503x over previous
import functools

import jax
import jax.numpy as jnp
from jax import lax
from jax.experimental import pallas as pl
from jax.experimental.pallas import tpu as pltpu
from jax.experimental.pallas import tpu_sc as plsc

H = 64
SENT = 200
BATCH = 4096
TOTAL = BATCH * SENT
EPS = 1e-5
NC = 2
NS = 16
NW = NC * NS
GRP = 128
GPC = 4
CHUNK = GRP * GPC
ROWS_PER_W = TOTAL // NW
NCHUNK = ROWS_PER_W // CHUNK

_mesh = plsc.VectorSubcoreMesh(core_axis_name="c", subcore_axis_name="s")


@functools.partial(
    pl.kernel,
    out_type=jax.ShapeDtypeStruct((TOTAL, H), jnp.float32),
    mesh=_mesh,
    scratch_types=[
        pltpu.VMEM((2, CHUNK), jnp.int32),
        pltpu.VMEM((2, CHUNK, H), jnp.float32),
        pltpu.VMEM((SENT, H), jnp.float32),
        pltpu.VMEM((H,), jnp.float32),
        pltpu.VMEM((H,), jnp.float32),
        pltpu.SemaphoreType.DMA,
        pltpu.SemaphoreType.DMA,
    ],
    compiler_params=pltpu.CompilerParams(
        needs_layout_passes=False, use_tc_tiling_on_sc=False),
)
def _sc_embed_ln(ids_hbm, tok_hbm, pos_hbm, w_hbm, b_hbm, out_hbm,
                 idx_v, rows_v, pos_v, w_v, b_v, gsem, osem):
    cid = lax.axis_index("c")
    sid = lax.axis_index("s")
    wid = sid * NC + cid
    base_w = wid * ROWS_PER_W

    pltpu.sync_copy(pos_hbm.at[pl.ds(0, SENT)], pos_v)
    pltpu.sync_copy(w_hbm, w_v)
    pltpu.sync_copy(b_hbm, b_v)

    def issue(c, b):
        off = pl.multiple_of(base_w + c * CHUNK, 8)
        pltpu.sync_copy(ids_hbm.at[pl.ds(off, CHUNK)], idx_v.at[b])
        for j in range(GPC):
            pltpu.async_copy(
                tok_hbm.at[idx_v.at[b, pl.ds(j * GRP, GRP)]],
                rows_v.at[b, pl.ds(j * GRP, GRP)], gsem)

    def drain_gathers(b):
        pltpu.make_async_copy(
            tok_hbm.at[idx_v.at[b, pl.ds(0, CHUNK)]], rows_v.at[b],
            gsem).wait()

    def drain_out():
        pltpu.make_async_copy(
            rows_v.at[0], out_hbm.at[pl.ds(0, CHUNK)], osem).wait()

    def compute(c, b):
        base = base_w + c * CHUNK

        @plsc.parallel_loop(0, CHUNK, 1, unroll=2)
        def row_loop(r):
            p = lax.rem(base + r, SENT)
            x = []
            for h in range(4):
                x.append(rows_v[b, r, pl.ds(16 * h, 16)]
                         + pos_v[p, pl.ds(16 * h, 16)])
            s = (x[0] + x[1]) + (x[2] + x[3])
            q = (x[0] * x[0] + x[1] * x[1]) + (x[2] * x[2] + x[3] * x[3])
            lanes = lax.iota(jnp.int32, 16)
            for m in (1, 2, 4, 8):
                perm = lanes ^ m
                s = s + s.at[perm].get(mode="promise_in_bounds")
                q = q + q.at[perm].get(mode="promise_in_bounds")
            mv = s * (1.0 / H)
            vv = q * (1.0 / H) - mv * mv + EPS
            iv = plsc.bitcast(vv, jnp.int32)
            y = plsc.bitcast(jnp.int32(0x5F3759DF) - (iv >> 1), jnp.float32)
            hv = vv * 0.5
            y = y * (1.5 - hv * y * y)
            y = y * (1.5 - hv * y * y)
            y = y * (1.5 - hv * y * y)
            for h in range(4):
                wgt = w_v[pl.ds(16 * h, 16)]
                bia = b_v[pl.ds(16 * h, 16)]
                rows_v[b, r, pl.ds(16 * h, 16)] = (x[h] - mv) * y * wgt + bia

    issue(0, 0)

    @pl.loop(0, NCHUNK // 2)
    def pair_loop(t):
        for b in range(2):
            c = t * 2 + b
            nb = 1 - b

            @pl.when(c + 1 < NCHUNK)
            def _():
                @pl.when(c >= 1)
                def _():
                    drain_out()
                issue(c + 1, nb)

            drain_gathers(b)
            compute(c, b)
            pltpu.async_copy(
                rows_v.at[b],
                out_hbm.at[pl.ds(base_w + c * CHUNK, CHUNK)], osem)

    drain_out()
    drain_out()


def kernel(batch_token_ids, token_table, pos_table, ln_weight, ln_bias):
    b, s = batch_token_ids.shape
    ids_flat = batch_token_ids.reshape(TOTAL).astype(jnp.int32)
    out = _sc_embed_ln(ids_flat, token_table, pos_table, ln_weight, ln_bias)
    return out.reshape(b, s, H)


# speedup vs baseline: 1.0871x; 1.0349x over previous
import functools

import jax
import jax.numpy as jnp
from jax import lax
from jax.experimental import pallas as pl
from jax.experimental.pallas import tpu as pltpu
from jax.experimental.pallas import tpu_sc as plsc

H = 64
SENT = 200
BATCH = 4096
TOTAL = BATCH * SENT
EPS = 1e-5
NC = 2
NS = 16
NW = NC * NS
GRP = 128
GPC = 4
CHUNK = GRP * GPC
ROWS_PER_W = TOTAL // NW
NCHUNK = ROWS_PER_W // CHUNK

_mesh = plsc.VectorSubcoreMesh(core_axis_name="c", subcore_axis_name="s")


@functools.partial(
    pl.kernel,
    out_type=jax.ShapeDtypeStruct((TOTAL, H), jnp.float32),
    mesh=_mesh,
    scratch_types=[
        pltpu.VMEM((2, CHUNK), jnp.int32),
        pltpu.VMEM((2, CHUNK, H), jnp.float32),
        pltpu.VMEM((SENT, H), jnp.float32),
        pltpu.VMEM((H,), jnp.float32),
        pltpu.VMEM((H,), jnp.float32),
        pltpu.SemaphoreType.DMA,
        pltpu.SemaphoreType.DMA,
    ],
    compiler_params=pltpu.CompilerParams(
        needs_layout_passes=False, use_tc_tiling_on_sc=False),
)
def _sc_embed_ln(ids_hbm, tok_hbm, pos_hbm, w_hbm, b_hbm, out_hbm,
                 idx_v, rows_v, pos_v, w_v, b_v, gsem, osem):
    cid = lax.axis_index("c")
    sid = lax.axis_index("s")
    wid = sid * NC + cid
    base_w = wid * ROWS_PER_W

    pltpu.sync_copy(pos_hbm.at[pl.ds(0, SENT)], pos_v)
    pltpu.sync_copy(w_hbm, w_v)
    pltpu.sync_copy(b_hbm, b_v)

    def issue(c, b):
        off = pl.multiple_of(base_w + c * CHUNK, 8)
        pltpu.sync_copy(ids_hbm.at[pl.ds(off, CHUNK)], idx_v.at[b])
        for j in range(GPC):
            pltpu.async_copy(
                tok_hbm.at[idx_v.at[b, pl.ds(j * GRP, GRP)]],
                rows_v.at[b, pl.ds(j * GRP, GRP)], gsem)

    def drain_gathers(b):
        pltpu.make_async_copy(
            tok_hbm.at[idx_v.at[b, pl.ds(0, CHUNK)]], rows_v.at[b],
            gsem).wait()

    def drain_out():
        pltpu.make_async_copy(
            rows_v.at[0], out_hbm.at[pl.ds(0, CHUNK)], osem).wait()

    def compute(c, b):
        base = base_w + c * CHUNK
        lanes = lax.iota(jnp.int32, 16)
        perms = [lanes ^ m for m in (1, 2, 4, 8)]
        wgt = [w_v[pl.ds(16 * h, 16)] for h in range(4)]
        bia = [b_v[pl.ds(16 * h, 16)] for h in range(4)]

        @plsc.parallel_loop(0, CHUNK, 1, unroll=4)
        def row_loop(r):
            p = lax.rem(base + r, SENT)
            x = []
            for h in range(4):
                x.append(rows_v[b, r, pl.ds(16 * h, 16)]
                         + pos_v[p, pl.ds(16 * h, 16)])
            s = (x[0] + x[1]) + (x[2] + x[3])
            q = (x[0] * x[0] + x[1] * x[1]) + (x[2] * x[2] + x[3] * x[3])
            for perm in perms:
                s = s + s.at[perm].get(mode="promise_in_bounds")
                q = q + q.at[perm].get(mode="promise_in_bounds")
            mv = s * (1.0 / H)
            vv = q * (1.0 / H) - mv * mv + EPS
            iv = plsc.bitcast(vv, jnp.int32)
            y = plsc.bitcast(jnp.int32(0x5F3759DF) - (iv >> 1), jnp.float32)
            hv = vv * 0.5
            y = y * (1.5 - hv * y * y)
            y = y * (1.5 - hv * y * y)
            my = mv * y
            for h in range(4):
                rows_v[b, r, pl.ds(16 * h, 16)] = (
                    (x[h] * y - my) * wgt[h] + bia[h])

    issue(0, 0)

    @pl.loop(0, NCHUNK // 2)
    def pair_loop(t):
        for b in range(2):
            c = t * 2 + b
            nb = 1 - b

            @pl.when(c + 1 < NCHUNK)
            def _():
                @pl.when(c >= 1)
                def _():
                    drain_out()
                issue(c + 1, nb)

            drain_gathers(b)
            compute(c, b)
            pltpu.async_copy(
                rows_v.at[b],
                out_hbm.at[pl.ds(base_w + c * CHUNK, CHUNK)], osem)

    drain_out()
    drain_out()


def kernel(batch_token_ids, token_table, pos_table, ln_weight, ln_bias):
    b, s = batch_token_ids.shape
    ids_flat = batch_token_ids.reshape(TOTAL).astype(jnp.int32)
    out = _sc_embed_ln(ids_flat, token_table, pos_table, ln_weight, ln_bias)
    return out.reshape(b, s, H)
